# Initial kernel scaffold; baseline (speedup 1.0000x reference)
import jax
import jax.numpy as jnp
from jax.experimental import pallas as pl


def kernel(points_xyz, points_fts):
    raise NotImplementedError("write your pallas kernel here")


# speedup vs baseline: 4.6375x; 4.6375x over previous
import functools

import jax
import jax.numpy as jnp
from jax import lax
from jax.experimental import pallas as pl
from jax.experimental.pallas import tpu as pltpu
from jax.experimental.pallas import tpu_sc as plsc

_B, _N, _C, _K = 4, 2048, 32, 16
_CD = _C * 3
_NK = _N * _K
_RB = 256

_NC, _NS = 2, 16
_NW = _NC * _NS
_WPB = _NW // _B
_RPW = _CD // _WPB


def _knn_body(xyz_ref, xyzT_ref, idx_ref):
    x = xyz_ref[0]
    xT = xyzT_ref[0]
    inner = lax.dot_general(
        x.astype(jnp.bfloat16), xT.astype(jnp.bfloat16),
        (((1,), (0,)), ((), ())),
        preferred_element_type=jnp.float32)
    sq_row = x[:, 0:1] ** 2 + x[:, 1:2] ** 2 + x[:, 2:3] ** 2
    sq_all = xT[0:1, :] ** 2 + xT[1:2, :] ** 2 + xT[2:3, :] ** 2
    dist = sq_row - 2.0 * inner + sq_all
    iota = lax.broadcasted_iota(jnp.int32, (_RB, _N), 1)
    cols = []
    for _ in range(_K):
        m = jnp.min(dist, axis=1, keepdims=True)
        am = jnp.min(jnp.where(dist == m, iota, _N),
                     axis=1, keepdims=True)
        cols.append(am)
        dist = jnp.where(iota == am, jnp.inf, dist)
    idx_ref[0] = jnp.concatenate(cols, axis=1)


def _knn_idx(points_xyz, xyzT):
    grid = (_B, _N // _RB)
    return pl.pallas_call(
        _knn_body,
        grid=grid,
        in_specs=[
            pl.BlockSpec((1, _RB, 3), lambda b, r: (b, r, 0)),
            pl.BlockSpec((1, 3, _N), lambda b, r: (b, 0, 0)),
        ],
        out_specs=pl.BlockSpec((1, _RB, _K), lambda b, r: (b, r, 0)),
        out_shape=jax.ShapeDtypeStruct((_B, _N, _K), jnp.int32),
    )(points_xyz, xyzT)


def _sc_body(fts_hbm, xyzT_hbm, idx_hbm, gf_hbm, gx_hbm,
             idx_v, in_v, diff_v, ctr_v):
    w = lax.axis_index("s") * _NC + lax.axis_index("c")
    b = w // _WPB
    slot = w % _WPB

    pltpu.sync_copy(idx_hbm.at[b], idx_v)

    def gather_row(write_ctr):
        def one(i, _):
            n0 = i * _K
            cvec = in_v[pl.ds(n0, _K)]
            for j in range(_K):
                iv = idx_v[pl.ds((n0 + j) * _K, _K)]
                g = plsc.load_gather(in_v, [iv])
                c = cvec[j]
                diff_v[pl.ds((n0 + j) * _K, _K)] = g - c
                if write_ctr:
                    ctr_v[pl.ds((n0 + j) * _K, _K)] = lax.broadcast(c, (_K,))
            return 0
        lax.fori_loop(0, _N // _K, one, 0)

    for t in range(_RPW):
        cd = slot * _RPW + t
        pltpu.sync_copy(fts_hbm.at[b, cd], in_v)
        gather_row(write_ctr=True)
        pltpu.sync_copy(diff_v, gf_hbm.at[b, 0, cd])
        pltpu.sync_copy(ctr_v, gf_hbm.at[b, 1, cd])

    @pl.when(slot < 3)
    def _():
        pltpu.sync_copy(xyzT_hbm.at[b, slot], in_v)
        gather_row(write_ctr=False)
        pltpu.sync_copy(diff_v, gx_hbm.at[b, slot])


def _sc_gather(fts, xyzT, idx):
    mesh = plsc.VectorSubcoreMesh(core_axis_name="c", subcore_axis_name="s",
                                  num_cores=_NC, num_subcores=_NS)
    f = pl.kernel(
        _sc_body,
        out_type=[
            jax.ShapeDtypeStruct((_B, 2, _CD, _NK), jnp.float32),
            jax.ShapeDtypeStruct((_B, 3, _NK), jnp.float32),
        ],
        mesh=mesh,
        compiler_params=pltpu.CompilerParams(needs_layout_passes=False),
        scratch_types=[
            pltpu.VMEM((_NK,), jnp.int32),
            pltpu.VMEM((_N,), jnp.float32),
            pltpu.VMEM((_NK,), jnp.float32),
            pltpu.VMEM((_NK,), jnp.float32),
        ],
    )
    return f(fts, xyzT, idx)


def kernel(points_xyz, points_fts):
    b, c, _, n = points_fts.shape
    xyzT = jnp.transpose(points_xyz, (0, 2, 1))
    idx = _knn_idx(points_xyz, xyzT)
    fts = points_fts.reshape(_B, _CD, _N)
    gf, gx = _sc_gather(fts, xyzT, idx.reshape(_B, _NK))
    group_fts = gf.reshape(_B, 2 * _C, 3, _N, _K)
    group_xyz = jnp.transpose(gx.reshape(_B, 3, _N, _K), (0, 2, 3, 1))
    new_fts = jnp.concatenate([points_fts, jnp.zeros_like(points_fts)],
                              axis=1)
    return (group_xyz, group_fts, points_xyz, new_fts)


# speedup vs baseline: 11.3623x; 2.4501x over previous
import jax
import jax.numpy as jnp
from jax import lax
from jax.experimental import pallas as pl
from jax.experimental.pallas import tpu as pltpu
from jax.experimental.pallas import tpu_sc as plsc

_B, _N, _C, _K = 4, 2048, 32, 16
_CD = _C * 3
_RB = 256

_NC, _NS = 2, 16
_NW = _NC * _NS
_WPB = _NW // _B
_RPW = _CD // _WPB


def _knn_body(xyz_ref, xyzT_ref, idxT_ref):
    xall = xyz_ref[0]
    xTr = xyzT_ref[0]
    innerT = lax.dot_general(
        xall.astype(jnp.bfloat16), xTr.astype(jnp.bfloat16),
        (((1,), (0,)), ((), ())),
        preferred_element_type=jnp.float32)
    sq_all = (xall[:, 0:1] ** 2 + xall[:, 1:2] ** 2
              + xall[:, 2:3] ** 2)
    sq_r = xTr[0:1, :] ** 2 + xTr[1:2, :] ** 2 + xTr[2:3, :] ** 2
    dist = sq_all - 2.0 * innerT + sq_r
    iota = lax.broadcasted_iota(jnp.int32, (_N, _RB), 0)
    rows = []
    for _ in range(_K):
        m = jnp.min(dist, axis=0, keepdims=True)
        am = jnp.min(jnp.where(dist == m, iota, _N),
                     axis=0, keepdims=True)
        rows.append(am)
        dist = jnp.where(iota == am, jnp.inf, dist)
    idxT_ref[0] = jnp.concatenate(rows, axis=0)


def _knn_idx(points_xyz, xyzT):
    grid = (_B, _N // _RB)
    return pl.pallas_call(
        _knn_body,
        grid=grid,
        in_specs=[
            pl.BlockSpec((1, _N, 3), lambda b, r: (b, 0, 0)),
            pl.BlockSpec((1, 3, _RB), lambda b, r: (b, 0, r)),
        ],
        out_specs=pl.BlockSpec((1, _K, _RB), lambda b, r: (b, 0, r)),
        out_shape=jax.ShapeDtypeStruct((_B, _K, _N), jnp.int32),
    )(points_xyz, xyzT)


def _sc_body(fts_hbm, xyzT_hbm, idxT_hbm, gf_hbm, gx_hbm,
             idx_v, in_v, diff_v):
    w = lax.axis_index("s") * _NC + lax.axis_index("c")
    b = w // _WPB
    slot = w % _WPB

    pltpu.sync_copy(idxT_hbm.at[b], idx_v)

    def gather_row():
        def one(i, _):
            n0 = i * 16
            cvec = in_v[pl.ds(n0, 16)]
            for k in range(_K):
                iv = idx_v[k, pl.ds(n0, 16)]
                g = plsc.load_gather(in_v, [iv])
                diff_v[k, pl.ds(n0, 16)] = g - cvec
            return 0
        lax.fori_loop(0, _N // 16, one, 0)

    for t in range(_RPW):
        cd = slot * _RPW + t
        pltpu.sync_copy(fts_hbm.at[b, cd], in_v)
        gather_row()
        pltpu.sync_copy(diff_v, gf_hbm.at[b, 0, cd])
        for k in range(_K):
            pltpu.sync_copy(in_v, gf_hbm.at[b, 1, cd, k])

    @pl.when(slot < 3)
    def _():
        pltpu.sync_copy(xyzT_hbm.at[b, slot], in_v)
        gather_row()
        pltpu.sync_copy(diff_v, gx_hbm.at[b, slot])


def _sc_gather(fts, xyzT, idxT):
    mesh = plsc.VectorSubcoreMesh(core_axis_name="c", subcore_axis_name="s",
                                  num_cores=_NC, num_subcores=_NS)
    f = pl.kernel(
        _sc_body,
        out_type=[
            jax.ShapeDtypeStruct((_B, 2, _CD, _K, _N), jnp.float32),
            jax.ShapeDtypeStruct((_B, 3, _K, _N), jnp.float32),
        ],
        mesh=mesh,
        compiler_params=pltpu.CompilerParams(needs_layout_passes=False),
        scratch_types=[
            pltpu.VMEM((_K, _N), jnp.int32),
            pltpu.VMEM((_N,), jnp.float32),
            pltpu.VMEM((_K, _N), jnp.float32),
        ],
    )
    return f(fts, xyzT, idxT)


def kernel(points_xyz, points_fts):
    xyzT = jnp.transpose(points_xyz, (0, 2, 1))
    idxT = _knn_idx(points_xyz, xyzT)
    fts = points_fts.reshape(_B, _CD, _N)
    gf, gx = _sc_gather(fts, xyzT, idxT)
    group_fts = jnp.transpose(gf.reshape(_B, 2 * _C, 3, _K, _N),
                              (0, 1, 2, 4, 3))
    group_xyz = jnp.transpose(gx, (0, 3, 2, 1))
    new_fts = jnp.concatenate([points_fts, jnp.zeros_like(points_fts)],
                              axis=1)
    return (group_xyz, group_fts, points_xyz, new_fts)


# speedup vs baseline: 12.4486x; 1.0956x over previous
import jax
import jax.numpy as jnp
from jax import lax
from jax.experimental import pallas as pl
from jax.experimental.pallas import tpu as pltpu
from jax.experimental.pallas import tpu_sc as plsc

_B, _N, _C, _K = 4, 2048, 32, 16
_CD = _C * 3
_RB = 256

_NC, _NS = 2, 16
_NW = _NC * _NS
_WPB = _NW // _B
_RPW = _CD // _WPB


def _knn_body(xyz_ref, xyzT_ref, idxT_ref):
    xall = xyz_ref[0]
    xTr = xyzT_ref[0]
    innerT = lax.dot_general(
        xall.astype(jnp.bfloat16), xTr.astype(jnp.bfloat16),
        (((1,), (0,)), ((), ())),
        preferred_element_type=jnp.float32)
    sq_all = (xall[:, 0:1] ** 2 + xall[:, 1:2] ** 2
              + xall[:, 2:3] ** 2)
    sq_r = xTr[0:1, :] ** 2 + xTr[1:2, :] ** 2 + xTr[2:3, :] ** 2
    dist = sq_all - 2.0 * innerT + sq_r
    iota = lax.broadcasted_iota(jnp.int32, (_N, _RB), 0)
    rows = []
    for _ in range(_K):
        m = jnp.min(dist, axis=0, keepdims=True)
        am = jnp.min(jnp.where(dist == m, iota, _N),
                     axis=0, keepdims=True)
        rows.append(am)
        dist = jnp.where(iota == am, jnp.inf, dist)
    idxT_ref[0] = jnp.concatenate(rows, axis=0)


def _knn_idx(points_xyz, xyzT):
    grid = (_B, _N // _RB)
    return pl.pallas_call(
        _knn_body,
        grid=grid,
        in_specs=[
            pl.BlockSpec((1, _N, 3), lambda b, r: (b, 0, 0)),
            pl.BlockSpec((1, 3, _RB), lambda b, r: (b, 0, r)),
        ],
        out_specs=pl.BlockSpec((1, _K, _RB), lambda b, r: (b, 0, r)),
        out_shape=jax.ShapeDtypeStruct((_B, _K, _N), jnp.int32),
    )(points_xyz, xyzT)


def _sc_body(fts_hbm, xyzT_hbm, idxT_hbm, gf_hbm, gx_hbm,
             idx_v, in_v, diff_v, sem_in, sem_d0, sem_d1,
             sem_c0, sem_c1, sem_c2):
    w = lax.axis_index("s") * _NC + lax.axis_index("c")
    b = w // _WPB
    slot = w % _WPB
    base = slot * _RPW
    sem_d = (sem_d0, sem_d1)
    sem_c = (sem_c0, sem_c1, sem_c2)

    pltpu.sync_copy(idxT_hbm.at[b], idx_v)

    def run_gather(bi3, bi2):
        def one(i, _):
            n0 = i * 16
            cvec = in_v[bi3, 0, pl.ds(n0, 16)]
            for k in range(_K):
                iv = idx_v[k, pl.ds(n0, 16)]
                g = plsc.load_gather(in_v.at[bi3, 0], [iv])
                diff_v[bi2, k, pl.ds(n0, 16)] = g - cvec
            return 0
        lax.fori_loop(0, _N // 16, one, 0)

    hin, hdiff, hctr = {}, {}, {}
    hin[0] = pltpu.async_copy(fts_hbm.at[b, pl.ds(base, 1)], in_v.at[0], sem_in)
    for t in range(_RPW):
        bi3, bi2 = t % 3, t % 2
        hin[t].wait()
        if t + 1 < _RPW:
            if t - 2 in hctr:
                for h in hctr.pop(t - 2):
                    h.wait()
            hin[t + 1] = pltpu.async_copy(
                fts_hbm.at[b, pl.ds(base + t + 1, 1)], in_v.at[(t + 1) % 3], sem_in)
        if t - 2 in hdiff:
            hdiff.pop(t - 2).wait()
        run_gather(bi3, bi2)
        hdiff[t] = pltpu.async_copy(
            diff_v.at[bi2], gf_hbm.at[b, 0, base + t], sem_d[bi2])
        hctr[t] = [
            pltpu.async_copy(in_v.at[bi3], gf_hbm.at[b, 1, base + t, pl.ds(k, 1)],
                             sem_c[bi3])
            for k in range(_K)]
    for t in sorted(hctr):
        for h in hctr[t]:
            h.wait()
    for t in sorted(hdiff):
        hdiff[t].wait()

    @pl.when(slot < 3)
    def _():
        pltpu.sync_copy(xyzT_hbm.at[b, pl.ds(slot, 1)], in_v.at[0])
        run_gather(0, 0)
        pltpu.sync_copy(diff_v.at[0], gx_hbm.at[b, slot])


def _sc_gather(fts, xyzT, idxT):
    mesh = plsc.VectorSubcoreMesh(core_axis_name="c", subcore_axis_name="s",
                                  num_cores=_NC, num_subcores=_NS)
    f = pl.kernel(
        _sc_body,
        out_type=[
            jax.ShapeDtypeStruct((_B, 2, _CD, _K, _N), jnp.float32),
            jax.ShapeDtypeStruct((_B, 3, _K, _N), jnp.float32),
        ],
        mesh=mesh,
        compiler_params=pltpu.CompilerParams(needs_layout_passes=False),
        scratch_types=[
            pltpu.VMEM((_K, _N), jnp.int32),
            pltpu.VMEM((3, 1, _N), jnp.float32),
            pltpu.VMEM((2, _K, _N), jnp.float32),
            pltpu.SemaphoreType.DMA,
            pltpu.SemaphoreType.DMA,
            pltpu.SemaphoreType.DMA,
            pltpu.SemaphoreType.DMA,
            pltpu.SemaphoreType.DMA,
            pltpu.SemaphoreType.DMA,
        ],
    )
    return f(fts, xyzT, idxT)


def kernel(points_xyz, points_fts):
    xyzT = jnp.transpose(points_xyz, (0, 2, 1))
    idxT = _knn_idx(points_xyz, xyzT)
    fts = points_fts.reshape(_B, _CD, _N)
    gf, gx = _sc_gather(fts, xyzT, idxT)
    group_fts = jnp.transpose(gf.reshape(_B, 2 * _C, 3, _K, _N),
                              (0, 1, 2, 4, 3))
    group_xyz = jnp.transpose(gx, (0, 3, 2, 1))
    new_fts = jnp.concatenate([points_fts, jnp.zeros_like(points_fts)],
                              axis=1)
    return (group_xyz, group_fts, points_xyz, new_fts)
